# Initial kernel scaffold; baseline (speedup 1.0000x reference)
import jax
import jax.numpy as jnp
from jax.experimental import pallas as pl


def kernel(x, edge_index, edge_weight, W1, b1, W2, b2):
    raise NotImplementedError("write your pallas kernel here")


# speedup vs baseline: 11.9581x; 11.9581x over previous
import functools

import jax
import jax.numpy as jnp
from jax import lax
from jax.experimental import pallas as pl
from jax.experimental.pallas import tpu as pltpu
from jax.experimental.pallas import tpu_sc as plsc

_N = 10000
_E = 320000
_D = 128
_NC = 2
_NS = 16
_NW = _NC * _NS
_C = 128
_K = -(-(_E // _NW) // _C)
_EPT = _K * _C
_EPAD = _NW * _EPT
_RPT = 632
_NPAD = _NS * _RPT
_BLK = 632
_G = _NPAD // _BLK

_f32 = jnp.float32
_i32 = jnp.int32


def _sc_mesh():
    return plsc.VectorSubcoreMesh(
        core_axis_name="c", subcore_axis_name="s",
        num_cores=_NC, num_subcores=_NS)


def _deg_kernel(dst3, ew3):
    def body(dst3_hbm, ew3_hbm, degp_hbm, dst_v, ew_v, zbuf, deg_sh):
        cid = lax.axis_index("c")
        sid = lax.axis_index("s")
        wid = cid * _NS + sid
        r0 = pl.multiple_of(sid * _RPT, 8)

        @pl.loop(0, 40)
        def zz(t):
            zbuf[pl.ds(t * 16, 16)] = jnp.zeros((16,), _f32)

        pltpu.sync_copy(zbuf.at[pl.ds(0, _RPT)], deg_sh.at[pl.ds(r0, _RPT)])
        pltpu.sync_copy(dst3_hbm.at[wid], dst_v)
        pltpu.sync_copy(ew3_hbm.at[wid], ew_v)
        plsc.subcore_barrier()

        @pl.loop(0, _K)
        def chunk(j):
            pltpu.sync_copy(ew_v.at[j], deg_sh.at[dst_v.at[j]], add=True)

        plsc.subcore_barrier()
        w0 = pl.multiple_of(cid * _NPAD + sid * _RPT, 8)
        pltpu.sync_copy(deg_sh.at[pl.ds(r0, _RPT)], zbuf.at[pl.ds(0, _RPT)])
        pltpu.sync_copy(zbuf.at[pl.ds(0, _RPT)], degp_hbm.at[pl.ds(w0, _RPT)])

    f = pl.kernel(
        body,
        out_type=jax.ShapeDtypeStruct((_NC * _NPAD,), _f32),
        mesh=_sc_mesh(),
        scratch_types=[
            pltpu.VMEM((_K, _C), _i32),
            pltpu.VMEM((_K, _C), _f32),
            pltpu.VMEM((640,), _f32),
            pltpu.VMEM_SHARED((_NPAD,), _f32),
        ],
    )
    return f(dst3, ew3)


def _edge_kernel(h, src3, dst3, ew3):
    def body(h_hbm, src3_hbm, dst3_hbm, ew3_hbm, sp_hbm,
             src_v, dst_v, ew_v, rows_v, acc_sh, sem):
        cid = lax.axis_index("c")
        sid = lax.axis_index("s")
        wid = cid * _NS + sid
        r0 = pl.multiple_of(sid * _RPT, 8)

        @pl.loop(0, _C)
        def zrow(e):
            for q in range(_D // 16):
                rows_v[e, pl.ds(q * 16, 16)] = jnp.zeros((16,), _f32)

        for t in range(4):
            pltpu.sync_copy(rows_v, acc_sh.at[pl.ds(r0 + t * _C, _C), :])
        pltpu.sync_copy(rows_v.at[pl.ds(0, _RPT - 4 * _C), :],
                        acc_sh.at[pl.ds(r0 + 4 * _C, _RPT - 4 * _C), :])
        pltpu.sync_copy(src3_hbm.at[wid], src_v)
        pltpu.sync_copy(dst3_hbm.at[wid], dst_v)
        pltpu.sync_copy(ew3_hbm.at[wid], ew_v)
        plsc.subcore_barrier()

        @pl.loop(0, _K)
        def chunk(j):
            pltpu.async_copy(h_hbm.at[src_v.at[j]], rows_v, sem).wait()

            @pl.loop(0, _C // 16)
            def grp(g):
                w16 = ew_v[j, pl.ds(g * 16, 16)]
                for l in range(16):
                    w = w16[jnp.full((16,), l, _i32)]
                    e = g * 16 + l
                    for q in range(_D // 16):
                        sl = pl.ds(q * 16, 16)
                        rows_v[e, sl] = rows_v[e, sl] * w

            pltpu.sync_copy(rows_v, acc_sh.at[dst_v.at[j]], add=True)

        plsc.subcore_barrier()
        for t in range(5):
            rows = _C if t < 4 else _RPT - 4 * _C
            pltpu.sync_copy(acc_sh.at[pl.ds(r0 + t * _C, rows), :],
                            rows_v.at[pl.ds(0, rows), :])
            pltpu.sync_copy(rows_v.at[pl.ds(0, rows), :],
                            sp_hbm.at[cid, pl.ds(r0 + t * _C, rows), :])

    f = pl.kernel(
        body,
        out_type=jax.ShapeDtypeStruct((_NC, _NPAD, _D), _f32),
        mesh=_sc_mesh(),
        scratch_types=[
            pltpu.VMEM((_K, _C), _i32),
            pltpu.VMEM((_K, _C), _i32),
            pltpu.VMEM((_K, _C), _f32),
            pltpu.VMEM((_C, _D), _f32),
            pltpu.VMEM_SHARED((_NPAD, _D), _f32),
            pltpu.SemaphoreType.DMA,
        ],
    )
    return f(h, src3, dst3, ew3)


def _m1_body(x_ref, w_ref, d0_ref, d1_ref, hp_ref, dinv_ref):
    dinv = lax.rsqrt(1.0 + d0_ref[...] + d1_ref[...])
    h = jnp.dot(x_ref[...], w_ref[...], preferred_element_type=_f32)
    hp_ref[...] = h * dinv
    dinv_ref[...] = dinv


def _tc_m1(xp, W1, d0, d1):
    return pl.pallas_call(
        _m1_body,
        grid=(_G,),
        in_specs=[
            pl.BlockSpec((_BLK, _D), lambda i: (i, 0)),
            pl.BlockSpec((_D, _D), lambda i: (0, 0)),
            pl.BlockSpec((_BLK, 1), lambda i: (i, 0)),
            pl.BlockSpec((_BLK, 1), lambda i: (i, 0)),
        ],
        out_specs=[
            pl.BlockSpec((_BLK, _D), lambda i: (i, 0)),
            pl.BlockSpec((_BLK, 1), lambda i: (i, 0)),
        ],
        out_shape=[
            jax.ShapeDtypeStruct((_NPAD, _D), _f32),
            jax.ShapeDtypeStruct((_NPAD, 1), _f32),
        ],
    )(xp, W1, d0, d1)


def _m2_body(s0_ref, s1_ref, hp_ref, dinv_ref, b_ref, w_ref, h2_ref):
    a = (s0_ref[...] + s1_ref[...] + hp_ref[...]) * dinv_ref[...] + b_ref[...]
    a = jnp.maximum(a, 0.0)
    h2 = jnp.dot(a, w_ref[...], preferred_element_type=_f32)
    h2_ref[...] = h2 * dinv_ref[...]


def _tc_m2(s0, s1, hp, dinv, b, W2):
    return pl.pallas_call(
        _m2_body,
        grid=(_G,),
        in_specs=[
            pl.BlockSpec((_BLK, _D), lambda i: (i, 0)),
            pl.BlockSpec((_BLK, _D), lambda i: (i, 0)),
            pl.BlockSpec((_BLK, _D), lambda i: (i, 0)),
            pl.BlockSpec((_BLK, 1), lambda i: (i, 0)),
            pl.BlockSpec((1, _D), lambda i: (0, 0)),
            pl.BlockSpec((_D, _D), lambda i: (0, 0)),
        ],
        out_specs=pl.BlockSpec((_BLK, _D), lambda i: (i, 0)),
        out_shape=jax.ShapeDtypeStruct((_NPAD, _D), _f32),
    )(s0, s1, hp, dinv, b, W2)


def _m3_body(s0_ref, s1_ref, hp_ref, dinv_ref, b_ref, out_ref):
    out_ref[...] = ((s0_ref[...] + s1_ref[...] + hp_ref[...])
                    * dinv_ref[...] + b_ref[...])


def _tc_m3(s0, s1, hp, dinv, b):
    return pl.pallas_call(
        _m3_body,
        grid=(_G,),
        in_specs=[
            pl.BlockSpec((_BLK, _D), lambda i: (i, 0)),
            pl.BlockSpec((_BLK, _D), lambda i: (i, 0)),
            pl.BlockSpec((_BLK, _D), lambda i: (i, 0)),
            pl.BlockSpec((_BLK, 1), lambda i: (i, 0)),
            pl.BlockSpec((1, _D), lambda i: (0, 0)),
        ],
        out_specs=pl.BlockSpec((_BLK, _D), lambda i: (i, 0)),
        out_shape=jax.ShapeDtypeStruct((_NPAD, _D), _f32),
    )(s0, s1, hp, dinv, b)


def kernel(x, edge_index, edge_weight, W1, b1, W2, b2):
    src = edge_index[0].astype(_i32)
    dst = edge_index[1].astype(_i32)
    ew = edge_weight.astype(_f32)
    pad = _EPAD - _E
    src3 = jnp.concatenate([src, jnp.zeros((pad,), _i32)]).reshape(_NW, _K, _C)
    dst3 = jnp.concatenate([dst, jnp.zeros((pad,), _i32)]).reshape(_NW, _K, _C)
    ew3 = jnp.concatenate([ew, jnp.zeros((pad,), _f32)]).reshape(_NW, _K, _C)
    xp = jnp.pad(x, ((0, _NPAD - _N), (0, 0)))

    degp = _deg_kernel(dst3, ew3).reshape(_NC, _NPAD)
    d0 = degp[0].reshape(_NPAD, 1)
    d1 = degp[1].reshape(_NPAD, 1)
    h1p, dinv = _tc_m1(xp, W1, d0, d1)
    s1 = _edge_kernel(h1p, src3, dst3, ew3)
    h2p = _tc_m2(s1[0], s1[1], h1p, dinv, b1.reshape(1, _D), W2)
    s2 = _edge_kernel(h2p, src3, dst3, ew3)
    outp = _tc_m3(s2[0], s2[1], h2p, dinv, b2.reshape(1, _D))
    return outp[:_N]
